# baseline (speedup 1.0000x reference)
import functools

import jax
import jax.numpy as jnp
import numpy as np
from jax.experimental import pallas as pl
from jax.experimental.pallas import tpu as pltpu

K = 20000
NUM_PTS = 5
EMBED_DIMS = 512
SEM_DIM = 256
CONV_OUT = 224
C1 = 112
NUM_FREQS = 8
PE_DIM = 4 * NUM_FREQS
LANE = 128
PEW = NUM_PTS * LANE
EPS = 1e-5

_SIN_C = (6.283185306916477, -41.34170218697257, 81.60524612664669,
          -76.70577668841639, 42.05753478200239, -15.085472586632998,
          3.778549078955688, -0.6179743754452339)


def _fused_kernel(g14_ref, hw_ref, lanes_ref, width_ref, city_ref,
                  scale14_ref, shift14_ref,
                  w1big_ref, b1t_ref, w2cat_ref, b2_ref, clng_ref, clnb_ref,
                  s640_ref, fp_ref, phq_ref, mask224_ref,
                  hwtab_ref, citytab_ref,
                  lw1_ref, lb1_ref, lw2_ref, lb2_ref, lmask_ref,
                  ww1_ref, wb1_ref, ww2_ref, wb2_ref, wmask_ref,
                  slng_ref, slnb_ref,
                  out_ref, coords_ref, *, block_b):
    B = block_b
    f32 = jnp.float32
    cpad = g14_ref[...] * scale14_ref[...] + shift14_ref[...]
    coords_ref[...] = cpad[:, 2:12]

    hw_ids = hw_ref[...]
    city_ids = city_ref[...]
    oh_hw = (hw_ids == jax.lax.broadcasted_iota(jnp.int32, (B, 12), 1)
             ).astype(f32)
    oh_city = (city_ids == jax.lax.broadcasted_iota(jnp.int32, (B, 4), 1)
               ).astype(f32)
    hw_feat = jnp.dot(oh_hw, hwtab_ref[...], preferred_element_type=f32)
    city_feat = jnp.dot(oh_city, citytab_ref[...], preferred_element_type=f32)

    lanes_i = lanes_ref[...]
    l1 = jax.nn.relu(lanes_i.astype(f32) * lw1_ref[...] + lb1_ref[...])
    lanes_proj = jnp.dot(l1, lw2_ref[...],
                         preferred_element_type=f32) + lb2_ref[...]
    lanes_feat = jnp.where(lanes_i != -1, lanes_proj, lmask_ref[...])

    width_f = width_ref[...]
    w1 = jax.nn.relu(width_f * ww1_ref[...] + wb1_ref[...])
    width_proj = jnp.dot(w1, ww2_ref[...],
                         preferred_element_type=f32) + wb2_ref[...]
    width_feat = jnp.where(width_f != -1.0, width_proj, wmask_ref[...])

    s = hw_feat + city_feat + lanes_feat + width_feat
    sm = jnp.mean(s, axis=-1, keepdims=True)
    sd = s - sm
    sv = jnp.mean(sd * sd, axis=-1, keepdims=True)
    sem = sd * jax.lax.rsqrt(sv + EPS) * slng_ref[...] + slnb_ref[...]

    y1p = jax.nn.relu(
        jnp.dot(cpad, w1big_ref[...], preferred_element_type=f32)
        + b1t_ref[...])
    y2all = jnp.dot(y1p, w2cat_ref[...], preferred_element_type=f32)

    chalf = cpad * 0.5
    c2 = (chalf - jnp.floor(chalf)) * 2.0
    cse = jnp.dot(c2, s640_ref[...], preferred_element_type=f32,
                  precision=jax.lax.Precision.HIGHEST)
    t = cse * fp_ref[...] + phq_ref[...]
    r = t - jnp.floor(t + 0.5)
    u2 = r * r
    poly = jnp.float32(_SIN_C[7])
    for c in _SIN_C[6::-1]:
        poly = poly * u2 + jnp.float32(c)
    pe640 = r * poly

    for p in range(NUM_PTS):
        x2 = jax.nn.relu(y2all[:, SEM_DIM * p:SEM_DIM * (p + 1)]
                         + b2_ref[...])
        m = jnp.sum(x2, axis=-1, keepdims=True) * (1.0 / CONV_OUT)
        d0 = (x2 - m) * mask224_ref[...]
        v = jnp.sum(d0 * d0, axis=-1, keepdims=True) * (1.0 / CONV_OUT)
        xln = d0 * jax.lax.rsqrt(v + EPS) * clng_ref[...] + clnb_ref[...]
        base = EMBED_DIMS * p
        out_ref[:, base:base + LANE] = (
            xln[:, :LANE] + pe640[:, LANE * p:LANE * (p + 1)])
        out_ref[:, base + LANE:base + SEM_DIM] = xln[:, LANE:]
        out_ref[:, base + SEM_DIM:base + EMBED_DIMS] = sem


@jax.jit
def kernel(geoms, highway_class, lanes, width, city,
           conv1_w, conv1_b, conv2_w, conv2_b, conv_ln_g, conv_ln_b,
           hw_table, city_table,
           lanes_w1, lanes_b1, lanes_w2, lanes_b2, lanes_mask,
           width_w1, width_b1, width_w2, width_b2, width_mask,
           sem_ln_g, sem_ln_b):
    B = 1000
    grid = K // B

    g14 = jnp.pad(geoms.reshape(K, 2 * NUM_PTS), ((0, 0), (2, 2)))
    sx, tx = 1.0 / 60.0, 0.5
    sy, ty = 1.0 / 30.0, 0.5
    scale14 = jnp.array([0.0, 0.0] + [sx, sy] * NUM_PTS + [0.0, 0.0],
                        jnp.float32).reshape(1, 14)
    shift14 = jnp.array([0.0, 0.0] + [tx, ty] * NUM_PTS + [0.0, 0.0],
                        jnp.float32).reshape(1, 14)

    w1im = jnp.transpose(conv1_w, (2, 1, 0)).reshape(6, C1)
    w1big = jnp.zeros((14, PEW), jnp.float32)
    b1t = jnp.zeros((1, PEW), jnp.float32)
    for p in range(NUM_PTS):
        w1big = jax.lax.dynamic_update_slice(w1big, w1im, (2 * p, LANE * p))
        b1t = jax.lax.dynamic_update_slice(
            b1t, conv1_b.reshape(1, C1), (0, LANE * p))
    w2t = jnp.transpose(conv2_w, (2, 1, 0))
    w2cat = jnp.zeros((PEW, NUM_PTS * SEM_DIM), jnp.float32)
    for p in range(NUM_PTS):
        for q in range(max(0, p - 1), min(NUM_PTS, p + 2)):
            w2cat = jax.lax.dynamic_update_slice(
                w2cat, w2t[q - p + 1], (LANE * q, SEM_DIM * p + PE_DIM))
    pad_row = lambda a: jnp.pad(a.reshape(1, -1), ((0, 0), (PE_DIM, 0)))

    s640 = np.zeros((14, PEW), np.float32)
    fp = np.zeros((1, PEW), np.float32)
    phq = np.zeros((1, PEW), np.float32)
    for p in range(NUM_PTS):
        base = LANE * p
        s640[2 + 2 * p, base:base + 16] = 1.0
        s640[3 + 2 * p, base + 16:base + 32] = 1.0
        for j in range(NUM_FREQS):
            for g in range(4):
                fp[0, base + 8 * g + j] = 2.0 ** (j - 1)
        for g in (1, 3):
            phq[0, base + 8 * g:base + 8 * g + 8] = 0.25
    mask224 = np.zeros((1, SEM_DIM), np.float32)
    mask224[0, PE_DIM:] = 1.0

    row = lambda a: a.reshape(1, -1)
    col_i = lambda a: a.reshape(K, 1).astype(jnp.int32)

    args = [
        g14,
        col_i(highway_class), col_i(lanes),
        width.reshape(K, 1).astype(jnp.float32), col_i(city),
        scale14, shift14,
        w1big, b1t, w2cat,
        pad_row(conv2_b), pad_row(conv_ln_g), pad_row(conv_ln_b),
        jnp.asarray(s640), jnp.asarray(fp), jnp.asarray(phq),
        jnp.asarray(mask224),
        hw_table, city_table,
        row(lanes_w1), row(lanes_b1), lanes_w2, row(lanes_b2),
        row(lanes_mask),
        row(width_w1), row(width_b1), width_w2, row(width_b2),
        row(width_mask),
        row(sem_ln_g), row(sem_ln_b),
    ]
    full = lambda a: pl.BlockSpec(a.shape, lambda i: (0,) * a.ndim)
    in_specs = [
        pl.BlockSpec((B, 14), lambda i: (i, 0)),
        pl.BlockSpec((B, 1), lambda i: (i, 0)),
        pl.BlockSpec((B, 1), lambda i: (i, 0)),
        pl.BlockSpec((B, 1), lambda i: (i, 0)),
        pl.BlockSpec((B, 1), lambda i: (i, 0)),
    ] + [full(a) for a in args[5:]]

    feat, coords = pl.pallas_call(
        functools.partial(_fused_kernel, block_b=B),
        grid=(grid,),
        compiler_params=pltpu.CompilerParams(
            dimension_semantics=("parallel",)),
        in_specs=in_specs,
        out_specs=[
            pl.BlockSpec((B, NUM_PTS * EMBED_DIMS), lambda i: (i, 0)),
            pl.BlockSpec((B, 2 * NUM_PTS), lambda i: (i, 0)),
        ],
        out_shape=[
            jax.ShapeDtypeStruct((K, NUM_PTS * EMBED_DIMS), jnp.float32),
            jax.ShapeDtypeStruct((K, 2 * NUM_PTS), jnp.float32),
        ],
    )(*args)

    sd_features = feat.reshape(1, K * NUM_PTS, EMBED_DIMS)
    sd_coords = coords.reshape(1, K * NUM_PTS, 2)
    sd_padding_mask = jnp.zeros((1, K * NUM_PTS), dtype=bool)
    return (sd_features, sd_padding_mask, sd_coords)


# speedup vs baseline: 1.0665x; 1.0665x over previous
import functools

import jax
import jax.numpy as jnp
import numpy as np
from jax.experimental import pallas as pl
from jax.experimental.pallas import tpu as pltpu

K = 20000
NUM_PTS = 5
EMBED_DIMS = 512
SEM_DIM = 256
CONV_OUT = 224
C1 = 112
NUM_FREQS = 8
PE_DIM = 4 * NUM_FREQS
LANE = 128
PEW = NUM_PTS * LANE
EPS = 1e-5

_SIN_C = (6.283185306916477, -41.34170218697257, 81.60524612664669,
          -76.70577668841639, 42.05753478200239, -15.085472586632998,
          3.778549078955688, -0.6179743754452339)


def _fused_kernel(g14_ref, hw_ref, lanes_ref, width_ref, city_ref,
                  scale14_ref, shift14_ref,
                  w1big_ref, b1t_ref, w2cat_ref, b2_ref, clng_ref, clnb_ref,
                  s640_ref, fp_ref, phq_ref, mask224_ref,
                  hwtab_ref, citytab_ref,
                  lw1_ref, lb1_ref, lw2_ref, lb2_ref, lmask_ref,
                  ww1_ref, wb1_ref, ww2_ref, wb2_ref, wmask_ref,
                  slng_ref, slnb_ref,
                  out_ref, coords_ref, *, block_b):
    B = block_b
    f32 = jnp.float32
    cpad = g14_ref[...] * scale14_ref[...] + shift14_ref[...]
    coords_ref[...] = cpad[:, 2:12]

    hw_ids = hw_ref[...]
    city_ids = city_ref[...]
    oh_hw = (hw_ids == jax.lax.broadcasted_iota(jnp.int32, (B, 12), 1)
             ).astype(f32)
    oh_city = (city_ids == jax.lax.broadcasted_iota(jnp.int32, (B, 4), 1)
               ).astype(f32)
    hw_feat = jnp.dot(oh_hw, hwtab_ref[...], preferred_element_type=f32)
    city_feat = jnp.dot(oh_city, citytab_ref[...], preferred_element_type=f32)

    lanes_i = lanes_ref[...]
    l1 = jax.nn.relu(lanes_i.astype(f32) * lw1_ref[...] + lb1_ref[...])
    lanes_proj = jnp.dot(l1, lw2_ref[...],
                         preferred_element_type=f32) + lb2_ref[...]
    lanes_feat = jnp.where(lanes_i != -1, lanes_proj, lmask_ref[...])

    width_f = width_ref[...]
    w1 = jax.nn.relu(width_f * ww1_ref[...] + wb1_ref[...])
    width_proj = jnp.dot(w1, ww2_ref[...],
                         preferred_element_type=f32) + wb2_ref[...]
    width_feat = jnp.where(width_f != -1.0, width_proj, wmask_ref[...])

    s = hw_feat + city_feat + lanes_feat + width_feat
    sm = jnp.mean(s, axis=-1, keepdims=True)
    sd = s - sm
    sv = jnp.mean(sd * sd, axis=-1, keepdims=True)
    sem = sd * jax.lax.rsqrt(sv + EPS) * slng_ref[...] + slnb_ref[...]

    y1p = jax.nn.relu(
        jnp.dot(cpad, w1big_ref[...], preferred_element_type=f32)
        + b1t_ref[...])
    z128 = jnp.zeros((B, LANE), f32)
    y1full = jnp.concatenate([z128, y1p, z128], axis=-1)

    chalf = cpad * 0.5
    c2 = (chalf - jnp.floor(chalf)) * 2.0
    cse = jnp.dot(c2, s640_ref[...], preferred_element_type=f32,
                  precision=jax.lax.Precision.HIGHEST)
    t = cse * fp_ref[...] + phq_ref[...]
    r = t - jnp.floor(t + 0.5)
    u2 = r * r
    poly = jnp.float32(_SIN_C[7])
    for c in _SIN_C[6::-1]:
        poly = poly * u2 + jnp.float32(c)
    pe640 = r * poly

    for p in range(NUM_PTS):
        win = y1full[:, LANE * p:LANE * p + 3 * LANE]
        acc = jnp.dot(win, w2cat_ref[...], preferred_element_type=f32)
        x2 = jax.nn.relu(acc + b2_ref[...])
        m = jnp.sum(x2, axis=-1, keepdims=True) * (1.0 / CONV_OUT)
        d0 = (x2 - m) * mask224_ref[...]
        v = jnp.sum(d0 * d0, axis=-1, keepdims=True) * (1.0 / CONV_OUT)
        xln = d0 * jax.lax.rsqrt(v + EPS) * clng_ref[...] + clnb_ref[...]
        base = EMBED_DIMS * p
        out_ref[:, base:base + LANE] = (
            xln[:, :LANE] + pe640[:, LANE * p:LANE * (p + 1)])
        out_ref[:, base + LANE:base + SEM_DIM] = xln[:, LANE:]
        out_ref[:, base + SEM_DIM:base + EMBED_DIMS] = sem


@jax.jit
def kernel(geoms, highway_class, lanes, width, city,
           conv1_w, conv1_b, conv2_w, conv2_b, conv_ln_g, conv_ln_b,
           hw_table, city_table,
           lanes_w1, lanes_b1, lanes_w2, lanes_b2, lanes_mask,
           width_w1, width_b1, width_w2, width_b2, width_mask,
           sem_ln_g, sem_ln_b):
    B = 1000
    grid = K // B

    g14 = jnp.pad(geoms.reshape(K, 2 * NUM_PTS), ((0, 0), (2, 2)))
    sx, tx = 1.0 / 60.0, 0.5
    sy, ty = 1.0 / 30.0, 0.5
    scale14 = jnp.array([0.0, 0.0] + [sx, sy] * NUM_PTS + [0.0, 0.0],
                        jnp.float32).reshape(1, 14)
    shift14 = jnp.array([0.0, 0.0] + [tx, ty] * NUM_PTS + [0.0, 0.0],
                        jnp.float32).reshape(1, 14)

    w1im = jnp.transpose(conv1_w, (2, 1, 0)).reshape(6, C1)
    w1big = jnp.zeros((14, PEW), jnp.float32)
    b1t = jnp.zeros((1, PEW), jnp.float32)
    for p in range(NUM_PTS):
        w1big = jax.lax.dynamic_update_slice(w1big, w1im, (2 * p, LANE * p))
        b1t = jax.lax.dynamic_update_slice(
            b1t, conv1_b.reshape(1, C1), (0, LANE * p))
    w2t = jnp.transpose(conv2_w, (2, 1, 0))
    w2cat = jnp.zeros((3 * LANE, SEM_DIM), jnp.float32)
    for d in range(3):
        w2cat = jax.lax.dynamic_update_slice(
            w2cat, w2t[d], (LANE * d, PE_DIM))
    pad_row = lambda a: jnp.pad(a.reshape(1, -1), ((0, 0), (PE_DIM, 0)))

    s640 = np.zeros((14, PEW), np.float32)
    fp = np.zeros((1, PEW), np.float32)
    phq = np.zeros((1, PEW), np.float32)
    for p in range(NUM_PTS):
        base = LANE * p
        s640[2 + 2 * p, base:base + 16] = 1.0
        s640[3 + 2 * p, base + 16:base + 32] = 1.0
        for j in range(NUM_FREQS):
            for g in range(4):
                fp[0, base + 8 * g + j] = 2.0 ** (j - 1)
        for g in (1, 3):
            phq[0, base + 8 * g:base + 8 * g + 8] = 0.25
    mask224 = np.zeros((1, SEM_DIM), np.float32)
    mask224[0, PE_DIM:] = 1.0

    row = lambda a: a.reshape(1, -1)
    col_i = lambda a: a.reshape(K, 1).astype(jnp.int32)

    args = [
        g14,
        col_i(highway_class), col_i(lanes),
        width.reshape(K, 1).astype(jnp.float32), col_i(city),
        scale14, shift14,
        w1big, b1t, w2cat,
        pad_row(conv2_b), pad_row(conv_ln_g), pad_row(conv_ln_b),
        jnp.asarray(s640), jnp.asarray(fp), jnp.asarray(phq),
        jnp.asarray(mask224),
        hw_table, city_table,
        row(lanes_w1), row(lanes_b1), lanes_w2, row(lanes_b2),
        row(lanes_mask),
        row(width_w1), row(width_b1), width_w2, row(width_b2),
        row(width_mask),
        row(sem_ln_g), row(sem_ln_b),
    ]
    full = lambda a: pl.BlockSpec(a.shape, lambda i: (0,) * a.ndim)
    in_specs = [
        pl.BlockSpec((B, 14), lambda i: (i, 0)),
        pl.BlockSpec((B, 1), lambda i: (i, 0)),
        pl.BlockSpec((B, 1), lambda i: (i, 0)),
        pl.BlockSpec((B, 1), lambda i: (i, 0)),
        pl.BlockSpec((B, 1), lambda i: (i, 0)),
    ] + [full(a) for a in args[5:]]

    feat, coords = pl.pallas_call(
        functools.partial(_fused_kernel, block_b=B),
        grid=(grid,),
        compiler_params=pltpu.CompilerParams(
            dimension_semantics=("parallel",)),
        in_specs=in_specs,
        out_specs=[
            pl.BlockSpec((B, NUM_PTS * EMBED_DIMS), lambda i: (i, 0)),
            pl.BlockSpec((B, 2 * NUM_PTS), lambda i: (i, 0)),
        ],
        out_shape=[
            jax.ShapeDtypeStruct((K, NUM_PTS * EMBED_DIMS), jnp.float32),
            jax.ShapeDtypeStruct((K, 2 * NUM_PTS), jnp.float32),
        ],
    )(*args)

    sd_features = feat.reshape(1, K * NUM_PTS, EMBED_DIMS)
    sd_coords = coords.reshape(1, K * NUM_PTS, 2)
    sd_padding_mask = jnp.zeros((1, K * NUM_PTS), dtype=bool)
    return (sd_features, sd_padding_mask, sd_coords)


# speedup vs baseline: 1.0893x; 1.0214x over previous
import functools

import jax
import jax.numpy as jnp
import numpy as np
from jax.experimental import pallas as pl
from jax.experimental.pallas import tpu as pltpu

K = 20000
NUM_PTS = 5
EMBED_DIMS = 512
SEM_DIM = 256
CONV_OUT = 224
C1 = 112
NUM_FREQS = 8
PE_DIM = 4 * NUM_FREQS
LANE = 128
PEW = NUM_PTS * LANE
EPS = 1e-5

_SIN_C = (6.283183465946359, -41.341480313261854, 81.59765670699102,
          -76.59491552319034, 41.26987033307485, -12.372395737097674)


def _fused_kernel(g14_ref, hw_ref, lanes_ref, width_ref, city_ref,
                  scale14_ref, shift14_ref, one0_ref,
                  w1big_ref, w2cat_ref, b2_ref, clng_ref, clnb_ref,
                  s640_ref,
                  hwtab_ref, citytab_ref,
                  lw1_ref, lb1_ref, lw2_ref, lb2_ref, lmask_ref,
                  ww1_ref, wb1_ref, ww2_ref, wb2_ref, wmask_ref,
                  slng_ref, slnb_ref,
                  out_ref, coords_ref, *, block_b):
    B = block_b
    f32 = jnp.float32
    cpad = g14_ref[...] * scale14_ref[...] + shift14_ref[...]
    coords_ref[...] = cpad[:, 2:12]

    hw_ids = hw_ref[...]
    city_ids = city_ref[...]
    oh_hw = (hw_ids == jax.lax.broadcasted_iota(jnp.int32, (B, 12), 1)
             ).astype(f32)
    oh_city = (city_ids == jax.lax.broadcasted_iota(jnp.int32, (B, 4), 1)
               ).astype(f32)
    hw_feat = jnp.dot(oh_hw, hwtab_ref[...], preferred_element_type=f32)
    city_feat = jnp.dot(oh_city, citytab_ref[...], preferred_element_type=f32)

    lanes_i = lanes_ref[...]
    l1 = jax.nn.relu(lanes_i.astype(f32) * lw1_ref[...] + lb1_ref[...])
    lanes_proj = jnp.dot(l1, lw2_ref[...],
                         preferred_element_type=f32) + lb2_ref[...]
    lanes_feat = jnp.where(lanes_i != -1, lanes_proj, lmask_ref[...])

    width_f = width_ref[...]
    w1 = jax.nn.relu(width_f * ww1_ref[...] + wb1_ref[...])
    width_proj = jnp.dot(w1, ww2_ref[...],
                         preferred_element_type=f32) + wb2_ref[...]
    width_feat = jnp.where(width_f != -1.0, width_proj, wmask_ref[...])

    s = hw_feat + city_feat + lanes_feat + width_feat
    sm = jnp.mean(s, axis=-1, keepdims=True)
    sd = s - sm
    sv = jnp.mean(sd * sd, axis=-1, keepdims=True)
    sem = sd * jax.lax.rsqrt(sv + EPS) * slng_ref[...] + slnb_ref[...]

    one0 = one0_ref[...]
    y1p = jax.nn.relu(
        jnp.dot(cpad + one0, w1big_ref[...], preferred_element_type=f32))
    z128 = jnp.zeros((B, LANE), f32)
    y1full = jnp.concatenate([z128, y1p, z128], axis=-1)

    chalf = cpad * 0.5
    c2 = (chalf - jnp.floor(chalf)) * 2.0 + one0
    z = jnp.dot(c2, s640_ref[...], preferred_element_type=f32,
                precision=jax.lax.Precision.HIGHEST)
    r = (z - jnp.floor(z)) - 0.5
    u2 = r * r
    poly = jnp.float32(_SIN_C[5])
    for c in _SIN_C[4::-1]:
        poly = poly * u2 + jnp.float32(c)
    pe640 = r * poly

    for p in range(NUM_PTS):
        win = y1full[:, LANE * p:LANE * p + 3 * LANE]
        acc = jnp.dot(win, w2cat_ref[...], preferred_element_type=f32)
        x2 = jax.nn.relu(acc + b2_ref[...])
        m = jnp.sum(x2, axis=-1, keepdims=True) * (1.0 / CONV_OUT)
        v = (jnp.sum(x2 * x2, axis=-1, keepdims=True) * (1.0 / CONV_OUT)
             - m * m)
        xln = (x2 - m) * (jax.lax.rsqrt(v + EPS) * clng_ref[...]) \
            + clnb_ref[...]
        base = EMBED_DIMS * p
        out_ref[:, base:base + LANE] = (
            xln[:, :LANE] + pe640[:, LANE * p:LANE * (p + 1)])
        out_ref[:, base + LANE:base + SEM_DIM] = xln[:, LANE:]
        out_ref[:, base + SEM_DIM:base + EMBED_DIMS] = sem


@jax.jit
def kernel(geoms, highway_class, lanes, width, city,
           conv1_w, conv1_b, conv2_w, conv2_b, conv_ln_g, conv_ln_b,
           hw_table, city_table,
           lanes_w1, lanes_b1, lanes_w2, lanes_b2, lanes_mask,
           width_w1, width_b1, width_w2, width_b2, width_mask,
           sem_ln_g, sem_ln_b):
    B = 1000
    grid = K // B

    g14 = jnp.pad(geoms.reshape(K, 2 * NUM_PTS), ((0, 0), (2, 2)))
    sx, tx = 1.0 / 60.0, 0.5
    sy, ty = 1.0 / 30.0, 0.5
    scale14 = jnp.array([0.0, 0.0] + [sx, sy] * NUM_PTS + [0.0, 0.0],
                        jnp.float32).reshape(1, 14)
    shift14 = jnp.array([0.0, 0.0] + [tx, ty] * NUM_PTS + [0.0, 0.0],
                        jnp.float32).reshape(1, 14)

    w1im = jnp.transpose(conv1_w, (2, 1, 0)).reshape(6, C1)
    w1big = jnp.zeros((14, PEW), jnp.float32)
    for p in range(NUM_PTS):
        w1big = jax.lax.dynamic_update_slice(w1big, w1im, (2 * p, LANE * p))
        w1big = jax.lax.dynamic_update_slice(
            w1big, conv1_b.reshape(1, C1), (0, LANE * p))
    w2t = jnp.transpose(conv2_w, (2, 1, 0))
    w2cat = jnp.zeros((3 * LANE, SEM_DIM), jnp.float32)
    for d in range(3):
        w2cat = jax.lax.dynamic_update_slice(
            w2cat, w2t[d], (LANE * d, PE_DIM))
    pad_row = lambda a: jnp.pad(a.reshape(1, -1), ((0, 0), (PE_DIM, 0)))

    s640 = np.zeros((14, PEW), np.float32)
    for p in range(NUM_PTS):
        base = LANE * p
        for j in range(NUM_FREQS):
            for g in range(4):
                lane = base + 8 * g + j
                s640[2 + 2 * p + (g // 2), lane] = 2.0 ** (j - 1)
                s640[0, lane] = 0.5 + (0.25 if g % 2 else 0.0)
    one0 = np.zeros((1, 14), np.float32)
    one0[0, 0] = 1.0

    row = lambda a: a.reshape(1, -1)
    col_i = lambda a: a.reshape(K, 1).astype(jnp.int32)

    args = [
        g14,
        col_i(highway_class), col_i(lanes),
        width.reshape(K, 1).astype(jnp.float32), col_i(city),
        scale14, shift14, jnp.asarray(one0),
        w1big, w2cat,
        pad_row(conv2_b), pad_row(conv_ln_g), pad_row(conv_ln_b),
        jnp.asarray(s640),
        hw_table, city_table,
        row(lanes_w1), row(lanes_b1), lanes_w2, row(lanes_b2),
        row(lanes_mask),
        row(width_w1), row(width_b1), width_w2, row(width_b2),
        row(width_mask),
        row(sem_ln_g), row(sem_ln_b),
    ]
    full = lambda a: pl.BlockSpec(a.shape, lambda i: (0,) * a.ndim)
    in_specs = [
        pl.BlockSpec((B, 14), lambda i: (i, 0)),
        pl.BlockSpec((B, 1), lambda i: (i, 0)),
        pl.BlockSpec((B, 1), lambda i: (i, 0)),
        pl.BlockSpec((B, 1), lambda i: (i, 0)),
        pl.BlockSpec((B, 1), lambda i: (i, 0)),
    ] + [full(a) for a in args[5:]]

    feat, coords = pl.pallas_call(
        functools.partial(_fused_kernel, block_b=B),
        grid=(grid,),
        compiler_params=pltpu.CompilerParams(
            dimension_semantics=("parallel",)),
        in_specs=in_specs,
        out_specs=[
            pl.BlockSpec((B, NUM_PTS * EMBED_DIMS), lambda i: (i, 0)),
            pl.BlockSpec((B, 2 * NUM_PTS), lambda i: (i, 0)),
        ],
        out_shape=[
            jax.ShapeDtypeStruct((K, NUM_PTS * EMBED_DIMS), jnp.float32),
            jax.ShapeDtypeStruct((K, 2 * NUM_PTS), jnp.float32),
        ],
    )(*args)

    sd_features = feat.reshape(1, K * NUM_PTS, EMBED_DIMS)
    sd_coords = coords.reshape(1, K * NUM_PTS, 2)
    sd_padding_mask = jnp.zeros((1, K * NUM_PTS), dtype=bool)
    return (sd_features, sd_padding_mask, sd_coords)


# speedup vs baseline: 1.1832x; 1.0862x over previous
import functools

import jax
import jax.numpy as jnp
import numpy as np
from jax.experimental import pallas as pl
from jax.experimental.pallas import tpu as pltpu

K = 20000
NUM_PTS = 5
EMBED_DIMS = 512
SEM_DIM = 256
CONV_OUT = 224
C1 = 112
NUM_FREQS = 8
PE_DIM = 4 * NUM_FREQS
LANE = 128
PEW = NUM_PTS * LANE
EPS = 1e-5

_SIN_C = (6.283183465946359, -41.341480313261854, 81.59765670699102,
          -76.59491552319034, 41.26987033307485, -12.372395737097674)


def _fused_kernel(g18_ref,
                  scale14_ref, shift14_ref, one0_ref,
                  w1big_ref, w2cat_ref, b2_ref, clng_ref, clnb_ref,
                  s640_ref,
                  hwtab_ref, citytab_ref,
                  lw1_ref, lb1_ref, lw2_ref, lb2_ref, lmask_ref,
                  ww1_ref, wb1_ref, ww2_ref, wb2_ref, wmask_ref,
                  slng_ref, slnb_ref,
                  out_ref, coords_ref, *, block_b):
    B = block_b
    f32 = jnp.float32
    g18 = g18_ref[...]
    cpad = g18[:, :14] * scale14_ref[...] + shift14_ref[...]
    coords_ref[...] = cpad[:, 2:12]

    hw_ids = g18[:, 14:15]
    lanes_f = g18[:, 15:16]
    width_f = g18[:, 16:17]
    city_ids = g18[:, 17:18]
    oh_hw = (hw_ids == jax.lax.broadcasted_iota(
        jnp.int32, (B, 12), 1).astype(f32)).astype(f32)
    oh_city = (city_ids == jax.lax.broadcasted_iota(
        jnp.int32, (B, 4), 1).astype(f32)).astype(f32)
    hw_feat = jnp.dot(oh_hw, hwtab_ref[...], preferred_element_type=f32)
    city_feat = jnp.dot(oh_city, citytab_ref[...], preferred_element_type=f32)

    l1 = jax.nn.relu(lanes_f * lw1_ref[...] + lb1_ref[...])
    lanes_proj = jnp.dot(l1, lw2_ref[...],
                         preferred_element_type=f32) + lb2_ref[...]
    lanes_feat = jnp.where(lanes_f != -1.0, lanes_proj, lmask_ref[...])

    w1 = jax.nn.relu(width_f * ww1_ref[...] + wb1_ref[...])
    width_proj = jnp.dot(w1, ww2_ref[...],
                         preferred_element_type=f32) + wb2_ref[...]
    width_feat = jnp.where(width_f != -1.0, width_proj, wmask_ref[...])

    s = hw_feat + city_feat + lanes_feat + width_feat
    sm = jnp.mean(s, axis=-1, keepdims=True)
    sd = s - sm
    sv = jnp.mean(sd * sd, axis=-1, keepdims=True)
    sem = sd * jax.lax.rsqrt(sv + EPS) * slng_ref[...] + slnb_ref[...]

    one0 = one0_ref[...]
    y1p = jax.nn.relu(
        jnp.dot(cpad + one0, w1big_ref[...], preferred_element_type=f32))
    z128 = jnp.zeros((B, LANE), f32)
    y1full = jnp.concatenate([z128, y1p, z128], axis=-1)

    chalf = cpad * 0.5
    c2 = (chalf - jnp.floor(chalf)) * 2.0 + one0
    z = jnp.dot(c2, s640_ref[...], preferred_element_type=f32,
                precision=jax.lax.Precision.HIGHEST)
    r = (z - jnp.floor(z)) - 0.5
    u2 = r * r
    poly = jnp.float32(_SIN_C[5])
    for c in _SIN_C[4::-1]:
        poly = poly * u2 + jnp.float32(c)
    pe640 = r * poly

    for p in range(NUM_PTS):
        win = y1full[:, LANE * p:LANE * p + 3 * LANE]
        acc = jnp.dot(win, w2cat_ref[...], preferred_element_type=f32)
        x2 = jax.nn.relu(acc + b2_ref[...])
        m = jnp.sum(x2, axis=-1, keepdims=True) * (1.0 / CONV_OUT)
        v = (jnp.sum(x2 * x2, axis=-1, keepdims=True) * (1.0 / CONV_OUT)
             - m * m)
        xln = (x2 - m) * (jax.lax.rsqrt(v + EPS) * clng_ref[...]) \
            + clnb_ref[...]
        base = EMBED_DIMS * p
        out_ref[:, base:base + LANE] = (
            xln[:, :LANE] + pe640[:, LANE * p:LANE * (p + 1)])
        out_ref[:, base + LANE:base + SEM_DIM] = xln[:, LANE:]
        out_ref[:, base + SEM_DIM:base + EMBED_DIMS] = sem


@jax.jit
def kernel(geoms, highway_class, lanes, width, city,
           conv1_w, conv1_b, conv2_w, conv2_b, conv_ln_g, conv_ln_b,
           hw_table, city_table,
           lanes_w1, lanes_b1, lanes_w2, lanes_b2, lanes_mask,
           width_w1, width_b1, width_w2, width_b2, width_mask,
           sem_ln_g, sem_ln_b):
    B = 1000
    grid = K // B

    g18 = jnp.concatenate([
        jnp.pad(geoms.reshape(K, 2 * NUM_PTS), ((0, 0), (2, 2))),
        highway_class.astype(jnp.float32).reshape(K, 1),
        lanes.astype(jnp.float32).reshape(K, 1),
        width.astype(jnp.float32).reshape(K, 1),
        city.astype(jnp.float32).reshape(K, 1),
    ], axis=1)
    sx, tx = 1.0 / 60.0, 0.5
    sy, ty = 1.0 / 30.0, 0.5
    scale14 = jnp.array([0.0, 0.0] + [sx, sy] * NUM_PTS + [0.0, 0.0],
                        jnp.float32).reshape(1, 14)
    shift14 = jnp.array([0.0, 0.0] + [tx, ty] * NUM_PTS + [0.0, 0.0],
                        jnp.float32).reshape(1, 14)

    w1im = jnp.transpose(conv1_w, (2, 1, 0)).reshape(6, C1)
    w1big = jnp.zeros((14, PEW), jnp.float32)
    for p in range(NUM_PTS):
        w1big = jax.lax.dynamic_update_slice(w1big, w1im, (2 * p, LANE * p))
        w1big = jax.lax.dynamic_update_slice(
            w1big, conv1_b.reshape(1, C1), (0, LANE * p))
    w2t = jnp.transpose(conv2_w, (2, 1, 0))
    w2cat = jnp.zeros((3 * LANE, SEM_DIM), jnp.float32)
    for d in range(3):
        w2cat = jax.lax.dynamic_update_slice(
            w2cat, w2t[d], (LANE * d, PE_DIM))
    pad_row = lambda a: jnp.pad(a.reshape(1, -1), ((0, 0), (PE_DIM, 0)))

    s640 = np.zeros((14, PEW), np.float32)
    for p in range(NUM_PTS):
        base = LANE * p
        for j in range(NUM_FREQS):
            for g in range(4):
                lane = base + 8 * g + j
                s640[2 + 2 * p + (g // 2), lane] = 2.0 ** (j - 1)
                s640[0, lane] = 0.5 + (0.25 if g % 2 else 0.0)
    one0 = np.zeros((1, 14), np.float32)
    one0[0, 0] = 1.0

    row = lambda a: a.reshape(1, -1)

    args = [
        g18,
        scale14, shift14, jnp.asarray(one0),
        w1big, w2cat,
        pad_row(conv2_b), pad_row(conv_ln_g), pad_row(conv_ln_b),
        jnp.asarray(s640),
        hw_table, city_table,
        row(lanes_w1), row(lanes_b1), lanes_w2, row(lanes_b2),
        row(lanes_mask),
        row(width_w1), row(width_b1), width_w2, row(width_b2),
        row(width_mask),
        row(sem_ln_g), row(sem_ln_b),
    ]
    full = lambda a: pl.BlockSpec(a.shape, lambda i: (0,) * a.ndim)
    in_specs = [
        pl.BlockSpec((B, 18), lambda i: (i, 0)),
    ] + [full(a) for a in args[1:]]

    feat, coords = pl.pallas_call(
        functools.partial(_fused_kernel, block_b=B),
        grid=(grid,),
        compiler_params=pltpu.CompilerParams(
            dimension_semantics=("parallel",)),
        in_specs=in_specs,
        out_specs=[
            pl.BlockSpec((B, NUM_PTS * EMBED_DIMS), lambda i: (i, 0)),
            pl.BlockSpec((B, 2 * NUM_PTS), lambda i: (i, 0)),
        ],
        out_shape=[
            jax.ShapeDtypeStruct((K, NUM_PTS * EMBED_DIMS), jnp.float32),
            jax.ShapeDtypeStruct((K, 2 * NUM_PTS), jnp.float32),
        ],
    )(*args)

    sd_features = feat.reshape(1, K * NUM_PTS, EMBED_DIMS)
    sd_coords = coords.reshape(1, K * NUM_PTS, 2)
    sd_padding_mask = jnp.zeros((1, K * NUM_PTS), dtype=bool)
    return (sd_features, sd_padding_mask, sd_coords)
